# speedup vs baseline: 5.8948x; 5.8948x over previous
import jax
import jax.numpy as jnp
from jax.experimental import pallas as pl
from jax.experimental.pallas import tpu as pltpu

TOTAL = 32768
B = 16
IN_CH = 4
HID = 64
FEAT = 16
OUT = 256
P = 3.0
EPS = 1e-6

CHUNK = 4096
NUM = TOTAL // CHUNK


def _gem_kernel(feats_ref, ids_ref, w1_ref, w2_ref, wfc_ref, out_ref,
                seg_ref, cnt_ref):
    i = pl.program_id(0)

    @pl.when(i == 0)
    def _init():
        seg_ref[...] = jnp.zeros_like(seg_ref)
        cnt_ref[...] = jnp.zeros_like(cnt_ref)

    x = jnp.maximum(
        jnp.dot(feats_ref[...], w1_ref[...],
                preferred_element_type=jnp.float32), 0.0)
    x = jnp.dot(x, w2_ref[...], preferred_element_type=jnp.float32)
    xc = jnp.maximum(x, EPS)
    xp = xc * xc * xc

    ids = ids_ref[0, 0, :]
    onehot = (ids[:, None] == jax.lax.broadcasted_iota(
        jnp.int32, (CHUNK, B), 1)).astype(jnp.float32)
    seg_ref[...] += jax.lax.dot_general(
        onehot, xp, dimension_numbers=(((0,), (0,)), ((), ())),
        preferred_element_type=jnp.float32)
    cnt_ref[...] += jnp.sum(onehot, axis=0, keepdims=True)

    @pl.when(i == NUM - 1)
    def _finish():
        cnt = cnt_ref[0, :]
        max_len = jnp.max(cnt)
        pad = (max_len - cnt)[:, None] * (EPS ** 3)
        gem = jnp.power((seg_ref[...] + pad) / max_len, 1.0 / 3.0)
        out_ref[...] = jnp.dot(gem, wfc_ref[...],
                               preferred_element_type=jnp.float32)


@jax.jit
def kernel(feats, batch_ids, W1, W2, Wfc):
    ids3 = batch_ids.reshape(NUM, 1, CHUNK)
    return pl.pallas_call(
        _gem_kernel,
        grid=(NUM,),
        in_specs=[
            pl.BlockSpec((CHUNK, IN_CH), lambda i: (i, 0)),
            pl.BlockSpec((1, 1, CHUNK), lambda i: (i, 0, 0)),
            pl.BlockSpec((IN_CH, HID), lambda i: (0, 0)),
            pl.BlockSpec((HID, FEAT), lambda i: (0, 0)),
            pl.BlockSpec((FEAT, OUT), lambda i: (0, 0)),
        ],
        out_specs=pl.BlockSpec((B, OUT), lambda i: (0, 0)),
        out_shape=jax.ShapeDtypeStruct((B, OUT), jnp.float32),
        scratch_shapes=[
            pltpu.VMEM((B, FEAT), jnp.float32),
            pltpu.VMEM((1, B), jnp.float32),
        ],
    )(feats, ids3, W1, W2, Wfc)


# speedup vs baseline: 6.1756x; 1.0476x over previous
import jax
import jax.numpy as jnp
from jax.experimental import pallas as pl
from jax.experimental.pallas import tpu as pltpu

TOTAL = 32768
B = 16
IN_CH = 4
HID = 64
FEAT = 16
OUT = 256
P = 3.0
EPS = 1e-6

CHUNK = 8192
NUM = TOTAL // CHUNK


def _gem_kernel(feats_ref, ids_ref, w1_ref, w2_ref, wfc_ref, out_ref,
                seg_ref):
    i = pl.program_id(0)

    @pl.when(i == 0)
    def _init():
        seg_ref[...] = jnp.zeros_like(seg_ref)

    x = jnp.maximum(
        jnp.dot(feats_ref[...], w1_ref[...],
                preferred_element_type=jnp.float32), 0.0)
    x = jnp.dot(x, w2_ref[...], preferred_element_type=jnp.float32)
    xc = jnp.maximum(x, EPS)
    xp = xc * xc * xc
    xp_ext = jnp.pad(xp, ((0, 0), (0, 1)), constant_values=1.0)

    ids = ids_ref[0, 0, :]
    onehot = (ids[:, None] == jax.lax.broadcasted_iota(
        jnp.int32, (CHUNK, B), 1)).astype(jnp.float32)
    seg_ref[...] += jax.lax.dot_general(
        onehot, xp_ext, dimension_numbers=(((0,), (0,)), ((), ())),
        preferred_element_type=jnp.float32)

    @pl.when(i == NUM - 1)
    def _finish():
        cnt = seg_ref[:, FEAT]
        max_len = jnp.max(cnt)
        pad = (max_len - cnt)[:, None] * (EPS ** 3)
        gem = jnp.power((seg_ref[:, :FEAT] + pad) / max_len, 1.0 / 3.0)
        out_ref[...] = jnp.dot(gem, wfc_ref[...],
                               preferred_element_type=jnp.float32)


@jax.jit
def kernel(feats, batch_ids, W1, W2, Wfc):
    ids3 = batch_ids.reshape(NUM, 1, CHUNK)
    return pl.pallas_call(
        _gem_kernel,
        grid=(NUM,),
        in_specs=[
            pl.BlockSpec((CHUNK, IN_CH), lambda i: (i, 0)),
            pl.BlockSpec((1, 1, CHUNK), lambda i: (i, 0, 0)),
            pl.BlockSpec((IN_CH, HID), lambda i: (0, 0)),
            pl.BlockSpec((HID, FEAT), lambda i: (0, 0)),
            pl.BlockSpec((FEAT, OUT), lambda i: (0, 0)),
        ],
        out_specs=pl.BlockSpec((B, OUT), lambda i: (0, 0)),
        out_shape=jax.ShapeDtypeStruct((B, OUT), jnp.float32),
        scratch_shapes=[
            pltpu.VMEM((B, FEAT + 1), jnp.float32),
        ],
    )(feats, ids3, W1, W2, Wfc)


# speedup vs baseline: 9.0406x; 1.4639x over previous
import jax
import jax.numpy as jnp
from jax.experimental import pallas as pl
from jax.experimental.pallas import tpu as pltpu

TOTAL = 32768
B = 16
IN_CH = 4
HID = 64
FEAT = 16
OUT = 256
P = 3.0
EPS = 1e-6

CHUNK = 8192
NUM = TOTAL // CHUNK


def _gem_kernel(feats_ref, ids_ref, w1_ref, w2_ref, wfc_ref, out_ref,
                seg_ref):
    i = pl.program_id(0)

    @pl.when(i == 0)
    def _init():
        seg_ref[...] = jnp.zeros_like(seg_ref)

    x = jnp.maximum(
        jax.lax.dot_general(feats_ref[...], w1_ref[...],
                            dimension_numbers=(((0,), (0,)), ((), ())),
                            preferred_element_type=jnp.float32), 0.0)
    x = jnp.dot(x, w2_ref[...], preferred_element_type=jnp.float32)
    xc = jnp.maximum(x, EPS)
    xp = xc * xc * xc
    xp_ext = jnp.pad(xp, ((0, 0), (0, 1)), constant_values=1.0)

    ids = ids_ref[0, 0, :]
    onehot = (ids[:, None] == jax.lax.broadcasted_iota(
        jnp.int32, (CHUNK, B), 1)).astype(jnp.float32)
    seg_ref[...] += jax.lax.dot_general(
        onehot, xp_ext, dimension_numbers=(((0,), (0,)), ((), ())),
        preferred_element_type=jnp.float32)

    @pl.when(i == NUM - 1)
    def _finish():
        cnt = seg_ref[:, FEAT]
        max_len = jnp.max(cnt)
        pad = (max_len - cnt)[:, None] * (EPS ** 3)
        gem = jnp.power((seg_ref[:, :FEAT] + pad) / max_len, 1.0 / 3.0)
        out_ref[...] = jnp.dot(gem, wfc_ref[...],
                               preferred_element_type=jnp.float32)


@jax.jit
def kernel(feats, batch_ids, W1, W2, Wfc):
    ids3 = batch_ids.reshape(NUM, 1, CHUNK)
    featsT = feats.T
    return pl.pallas_call(
        _gem_kernel,
        grid=(NUM,),
        in_specs=[
            pl.BlockSpec((IN_CH, CHUNK), lambda i: (0, i)),
            pl.BlockSpec((1, 1, CHUNK), lambda i: (i, 0, 0)),
            pl.BlockSpec((IN_CH, HID), lambda i: (0, 0)),
            pl.BlockSpec((HID, FEAT), lambda i: (0, 0)),
            pl.BlockSpec((FEAT, OUT), lambda i: (0, 0)),
        ],
        out_specs=pl.BlockSpec((B, OUT), lambda i: (0, 0)),
        out_shape=jax.ShapeDtypeStruct((B, OUT), jnp.float32),
        scratch_shapes=[
            pltpu.VMEM((B, FEAT + 1), jnp.float32),
        ],
    )(featsT, ids3, W1, W2, Wfc)


# speedup vs baseline: 21.5317x; 2.3817x over previous
import jax
import jax.numpy as jnp
from jax.experimental import pallas as pl
from jax.experimental.pallas import tpu as pltpu

TOTAL = 32768
B = 16
IN_CH = 4
HID = 64
FEAT = 16
OUT = 256
P = 3.0
EPS = 1e-6

CHUNK = 8192
NUM = TOTAL // CHUNK


def _gem_kernel(featsT_ref, ids_ref, w1t_ref, w2t_ref, wfc_ref, out_ref,
                seg_ref):
    i = pl.program_id(0)

    @pl.when(i == 0)
    def _init():
        seg_ref[...] = jnp.zeros_like(seg_ref)

    x = jnp.maximum(
        jnp.dot(w1t_ref[...], featsT_ref[...],
                preferred_element_type=jnp.float32), 0.0)
    x = jnp.dot(w2t_ref[...], x,
                preferred_element_type=jnp.float32)
    xc = jnp.maximum(x, EPS)
    xp = xc * xc * xc
    xp_ext = jnp.pad(xp, ((0, 1), (0, 0)), constant_values=1.0)

    ids = ids_ref[0, 0, :]
    onehot = (ids[None, :] == jax.lax.broadcasted_iota(
        jnp.int32, (B, CHUNK), 0)).astype(jnp.float32)
    seg_ref[...] += jax.lax.dot_general(
        onehot, xp_ext, dimension_numbers=(((1,), (1,)), ((), ())),
        preferred_element_type=jnp.float32)

    @pl.when(i == NUM - 1)
    def _finish():
        cnt = seg_ref[:, FEAT]
        max_len = jnp.max(cnt)
        pad = (max_len - cnt)[:, None] * (EPS ** 3)
        gem = jnp.power((seg_ref[:, :FEAT] + pad) / max_len, 1.0 / 3.0)
        out_ref[...] = jnp.dot(gem, wfc_ref[...],
                               preferred_element_type=jnp.float32)


@jax.jit
def kernel(feats, batch_ids, W1, W2, Wfc):
    ids3 = batch_ids.reshape(NUM, 1, CHUNK)
    featsT = feats.T
    return pl.pallas_call(
        _gem_kernel,
        grid=(NUM,),
        in_specs=[
            pl.BlockSpec((IN_CH, CHUNK), lambda i: (0, i)),
            pl.BlockSpec((1, 1, CHUNK), lambda i: (i, 0, 0)),
            pl.BlockSpec((HID, IN_CH), lambda i: (0, 0)),
            pl.BlockSpec((FEAT, HID), lambda i: (0, 0)),
            pl.BlockSpec((FEAT, OUT), lambda i: (0, 0)),
        ],
        out_specs=pl.BlockSpec((B, OUT), lambda i: (0, 0)),
        out_shape=jax.ShapeDtypeStruct((B, OUT), jnp.float32),
        scratch_shapes=[
            pltpu.VMEM((B, FEAT + 1), jnp.float32),
        ],
    )(featsT, ids3, W1.T, W2.T, Wfc)


# speedup vs baseline: 23.7141x; 1.1014x over previous
import jax
import jax.numpy as jnp
from jax.experimental import pallas as pl
from jax.experimental.pallas import tpu as pltpu

TOTAL = 32768
B = 16
IN_CH = 4
HID = 64
FEAT = 16
OUT = 256
P = 3.0
EPS = 1e-6

CHUNK = 16384
NUM = TOTAL // CHUNK


def _gem_kernel(featsT_ref, ids_ref, w1t_ref, w2t_ref, wfc_ref, out_ref,
                seg_ref):
    i = pl.program_id(0)

    @pl.when(i == 0)
    def _init():
        seg_ref[...] = jnp.zeros_like(seg_ref)

    x = jnp.maximum(
        jnp.dot(w1t_ref[...], featsT_ref[...],
                preferred_element_type=jnp.float32), 0.0)
    x = jnp.dot(w2t_ref[...], x,
                preferred_element_type=jnp.float32)
    xc = jnp.maximum(x, EPS)
    xp = xc * xc * xc
    xp_ext = jnp.pad(xp, ((0, 1), (0, 0)), constant_values=1.0)

    ids = ids_ref[0, 0, :]
    onehot = (ids[None, :] == jax.lax.broadcasted_iota(
        jnp.int32, (B, CHUNK), 0)).astype(jnp.float32)
    seg_ref[...] += jax.lax.dot_general(
        onehot, xp_ext, dimension_numbers=(((1,), (1,)), ((), ())),
        preferred_element_type=jnp.float32)

    @pl.when(i == NUM - 1)
    def _finish():
        cnt = seg_ref[:, FEAT]
        max_len = jnp.max(cnt)
        pad = (max_len - cnt)[:, None] * (EPS ** 3)
        gem = jnp.power((seg_ref[:, :FEAT] + pad) / max_len, 1.0 / 3.0)
        out_ref[...] = jnp.dot(gem, wfc_ref[...],
                               preferred_element_type=jnp.float32)


@jax.jit
def kernel(feats, batch_ids, W1, W2, Wfc):
    ids3 = batch_ids.reshape(NUM, 1, CHUNK)
    featsT = feats.T
    return pl.pallas_call(
        _gem_kernel,
        grid=(NUM,),
        in_specs=[
            pl.BlockSpec((IN_CH, CHUNK), lambda i: (0, i)),
            pl.BlockSpec((1, 1, CHUNK), lambda i: (i, 0, 0)),
            pl.BlockSpec((HID, IN_CH), lambda i: (0, 0)),
            pl.BlockSpec((FEAT, HID), lambda i: (0, 0)),
            pl.BlockSpec((FEAT, OUT), lambda i: (0, 0)),
        ],
        out_specs=pl.BlockSpec((B, OUT), lambda i: (0, 0)),
        out_shape=jax.ShapeDtypeStruct((B, OUT), jnp.float32),
        scratch_shapes=[
            pltpu.VMEM((B, FEAT + 1), jnp.float32),
        ],
    )(featsT, ids3, W1.T, W2.T, Wfc)


# speedup vs baseline: 23.7297x; 1.0007x over previous
import jax
import jax.numpy as jnp
from jax.experimental import pallas as pl
from jax.experimental.pallas import tpu as pltpu

TOTAL = 32768
B = 16
IN_CH = 4
HID = 64
FEAT = 16
OUT = 256
P = 3.0
EPS = 1e-6

CHUNK = 32768
NUM = TOTAL // CHUNK


def _gem_kernel(featsT_ref, ids_ref, w1t_ref, w2t_ref, wfc_ref, out_ref,
                seg_ref):
    i = pl.program_id(0)

    @pl.when(i == 0)
    def _init():
        seg_ref[...] = jnp.zeros_like(seg_ref)

    x = jnp.maximum(
        jnp.dot(w1t_ref[...], featsT_ref[...],
                preferred_element_type=jnp.float32), 0.0)
    x = jnp.dot(w2t_ref[...], x,
                preferred_element_type=jnp.float32)
    xc = jnp.maximum(x, EPS)
    xp = xc * xc * xc
    xp_ext = jnp.pad(xp, ((0, 1), (0, 0)), constant_values=1.0)

    ids = ids_ref[0, 0, :]
    onehot = (ids[None, :] == jax.lax.broadcasted_iota(
        jnp.int32, (B, CHUNK), 0)).astype(jnp.float32)
    seg_ref[...] += jax.lax.dot_general(
        onehot, xp_ext, dimension_numbers=(((1,), (1,)), ((), ())),
        preferred_element_type=jnp.float32)

    @pl.when(i == NUM - 1)
    def _finish():
        cnt = seg_ref[:, FEAT]
        max_len = jnp.max(cnt)
        pad = (max_len - cnt)[:, None] * (EPS ** 3)
        gem = jnp.power((seg_ref[:, :FEAT] + pad) / max_len, 1.0 / 3.0)
        out_ref[...] = jnp.dot(gem, wfc_ref[...],
                               preferred_element_type=jnp.float32)


@jax.jit
def kernel(feats, batch_ids, W1, W2, Wfc):
    ids3 = batch_ids.reshape(NUM, 1, CHUNK)
    featsT = feats.T
    return pl.pallas_call(
        _gem_kernel,
        grid=(NUM,),
        in_specs=[
            pl.BlockSpec((IN_CH, CHUNK), lambda i: (0, i)),
            pl.BlockSpec((1, 1, CHUNK), lambda i: (i, 0, 0)),
            pl.BlockSpec((HID, IN_CH), lambda i: (0, 0)),
            pl.BlockSpec((FEAT, HID), lambda i: (0, 0)),
            pl.BlockSpec((FEAT, OUT), lambda i: (0, 0)),
        ],
        out_specs=pl.BlockSpec((B, OUT), lambda i: (0, 0)),
        out_shape=jax.ShapeDtypeStruct((B, OUT), jnp.float32),
        scratch_shapes=[
            pltpu.VMEM((B, FEAT + 1), jnp.float32),
        ],
    )(featsT, ids3, W1.T, W2.T, Wfc)
